# baseline (speedup 1.0000x reference)
import functools

import jax
import jax.numpy as jnp
from jax import lax
from jax.experimental import pallas as pl
from jax.experimental.pallas import tpu as pltpu
from jax.experimental.pallas import tpu_sc as plsc

_NC = 2
_NS = 16
_NW = _NC * _NS

_B = 16384
_N = _B + 1
_D = 64
_BPW = _B // _NW

_mesh = plsc.VectorSubcoreMesh(
    core_axis_name="c", subcore_axis_name="s", num_cores=_NC, num_subcores=_NS
)


@functools.partial(
    pl.kernel,
    mesh=_mesh,
    out_type=jax.ShapeDtypeStruct((2, _N, 1, _D), jnp.float32),
    compiler_params=pltpu.CompilerParams(use_tc_tiling_on_sc=False),
    scratch_types=[
        pltpu.VMEM((_BPW,), jnp.int32),
        pltpu.VMEM((_BPW, _D), jnp.float32),
        pltpu.VMEM((8,), jnp.int32),
        pltpu.VMEM((8, _D), jnp.float32),
        pltpu.SemaphoreType.DMA,
    ],
)
def _embed_lookup(idx_hbm, table_hbm, out_hbm, idx_v, rows_v, idx8_v, rows8_v, sem):
    wid = lax.axis_index("s") * _NC + lax.axis_index("c")
    base = wid * _BPW
    pltpu.sync_copy(idx_hbm.at[pl.ds(base, _BPW)], idx_v)
    pltpu.async_copy(table_hbm.at[idx_v], rows_v, sem).wait()
    pltpu.sync_copy(rows_v, out_hbm.at[0, pl.ds(base, _BPW), 0])
    pltpu.sync_copy(rows_v, out_hbm.at[1, pl.ds(base, _BPW), 0])

    @pl.when(wid == 0)
    def _():
        pltpu.sync_copy(idx_hbm.at[pl.ds(_B, 8)], idx8_v)
        pltpu.async_copy(table_hbm.at[idx8_v], rows8_v, sem).wait()
        pltpu.sync_copy(rows8_v.at[pl.ds(0, 1)], out_hbm.at[0, pl.ds(_B, 1), 0])
        pltpu.sync_copy(rows8_v.at[pl.ds(0, 1)], out_hbm.at[1, pl.ds(_B, 1), 0])


def kernel(inputs, embed_weight):
    idx = inputs.reshape(-1).astype(jnp.int32)
    padded_idx = jnp.concatenate(
        [jnp.zeros((1,), jnp.int32), idx, jnp.zeros((7,), jnp.int32)]
    )
    return _embed_lookup(padded_idx, embed_weight)


# speedup vs baseline: 2.0433x; 2.0433x over previous
import functools

import jax
import jax.numpy as jnp
from jax import lax
from jax.experimental import pallas as pl
from jax.experimental.pallas import tpu as pltpu
from jax.experimental.pallas import tpu_sc as plsc

_NC = 2
_NS = 16
_NW = _NC * _NS

_B = 16384
_N = _B + 1
_D = 64
_V = 32
_BPW = _B // _NW
_L = 16
_G = _BPW // _L

_mesh = plsc.VectorSubcoreMesh(
    core_axis_name="c", subcore_axis_name="s", num_cores=_NC, num_subcores=_NS
)


@functools.partial(
    pl.kernel,
    mesh=_mesh,
    out_type=jax.ShapeDtypeStruct((2, _D, _N), jnp.float32),
    compiler_params=pltpu.CompilerParams(needs_layout_passes=False),
    scratch_types=[
        pltpu.VMEM((_BPW + _L,), jnp.int32),
        pltpu.VMEM((_V, _D), jnp.float32),
        pltpu.VMEM((_D, _BPW + 1), jnp.float32),
    ],
)
def _embed_lookup(idx_hbm, table_hbm, out_hbm, idx_v, table_v, block_v):
    wid = lax.axis_index("s") * _NC + lax.axis_index("c")
    base = wid * _BPW
    pltpu.sync_copy(idx_hbm.at[pl.ds(base, _BPW + _L)], idx_v)
    pltpu.sync_copy(table_hbm, table_v)

    def body(g, carry):
        col = g * _L
        idx_vec = idx_v[pl.ds(col, _L)]
        for d in range(_D):
            colv = jnp.full((_L,), d, jnp.int32)
            block_v[d, pl.ds(col, _L)] = plsc.load_gather(table_v, [idx_vec, colv])
        return carry

    lax.fori_loop(0, _G, body, 0)

    @pl.when(wid == _NW - 1)
    def _():
        idx_vec = idx_v[pl.ds(_BPW, _L)]
        lane0 = lax.iota(jnp.int32, _L) == 0
        for d in range(_D):
            colv = jnp.full((_L,), d, jnp.int32)
            vals = plsc.load_gather(table_v, [idx_vec, colv])
            plsc.store_scatter(
                block_v,
                [colv, jnp.full((_L,), _BPW, jnp.int32)],
                vals,
                mask=lane0,
            )

    @pl.when(wid < _NW - 1)
    def _():
        pltpu.sync_copy(block_v.at[:, pl.ds(0, _BPW)], out_hbm.at[0, :, pl.ds(base, _BPW)])
        pltpu.sync_copy(block_v.at[:, pl.ds(0, _BPW)], out_hbm.at[1, :, pl.ds(base, _BPW)])

    @pl.when(wid == _NW - 1)
    def _():
        pltpu.sync_copy(block_v, out_hbm.at[0, :, pl.ds(_B - _BPW, _BPW + 1)])
        pltpu.sync_copy(block_v, out_hbm.at[1, :, pl.ds(_B - _BPW, _BPW + 1)])


def kernel(inputs, embed_weight):
    idx = inputs.reshape(-1).astype(jnp.int32)
    padded_idx = jnp.concatenate(
        [jnp.zeros((1,), jnp.int32), idx, jnp.zeros((15,), jnp.int32)]
    )
    out = _embed_lookup(padded_idx, embed_weight)
    return out.transpose(0, 2, 1).reshape(2, _N, 1, _D)


# speedup vs baseline: 2.5825x; 1.2639x over previous
import functools

import jax
import jax.numpy as jnp
from jax import lax
from jax.experimental import pallas as pl
from jax.experimental.pallas import tpu as pltpu
from jax.experimental.pallas import tpu_sc as plsc

_NC = 2
_NS = 16
_NW = _NC * _NS

_B = 16384
_N = _B + 1
_D = 64
_V = 32
_BPW = _B // _NW
_L = 16
_G = _BPW // _L

_mesh = plsc.VectorSubcoreMesh(
    core_axis_name="c", subcore_axis_name="s", num_cores=_NC, num_subcores=_NS
)


@functools.partial(
    pl.kernel,
    mesh=_mesh,
    out_type=jax.ShapeDtypeStruct((2, _D, _N), jnp.float32),
    compiler_params=pltpu.CompilerParams(needs_layout_passes=False),
    scratch_types=[
        pltpu.VMEM((_BPW + _L,), jnp.int32),
        pltpu.VMEM((_V, _D), jnp.float32),
        pltpu.VMEM((_D, _BPW + 1), jnp.float32),
        pltpu.SemaphoreType.DMA,
        pltpu.SemaphoreType.DMA,
    ],
)
def _embed_lookup(idx_hbm, table_hbm, out_hbm, idx_v, table_v, block_v, sem0, sem1):
    wid = lax.axis_index("s") * _NC + lax.axis_index("c")
    base = wid * _BPW
    cp_idx = pltpu.async_copy(idx_hbm.at[pl.ds(base, _BPW + _L)], idx_v, sem0)
    cp_tbl = pltpu.async_copy(table_hbm, table_v, sem1)
    cp_idx.wait()
    cp_tbl.wait()

    @plsc.parallel_loop(0, _G, 1)
    def _(g):
        col = g * _L
        idx_vec = idx_v[pl.ds(col, _L)]
        for d in range(_D):
            colv = jnp.full((_L,), d, jnp.int32)
            block_v[d, pl.ds(col, _L)] = plsc.load_gather(table_v, [idx_vec, colv])

    @pl.when(wid == _NW - 1)
    def _():
        idx_vec = idx_v[pl.ds(_BPW, _L)]
        lane0 = lax.iota(jnp.int32, _L) == 0
        for d in range(_D):
            colv = jnp.full((_L,), d, jnp.int32)
            vals = plsc.load_gather(table_v, [idx_vec, colv])
            plsc.store_scatter(
                block_v,
                [colv, jnp.full((_L,), _BPW, jnp.int32)],
                vals,
                mask=lane0,
            )

    @pl.when(wid < _NW - 1)
    def _():
        w0 = pltpu.async_copy(
            block_v.at[:, pl.ds(0, _BPW)], out_hbm.at[0, :, pl.ds(base, _BPW)], sem0
        )
        w1 = pltpu.async_copy(
            block_v.at[:, pl.ds(0, _BPW)], out_hbm.at[1, :, pl.ds(base, _BPW)], sem1
        )
        w0.wait()
        w1.wait()

    @pl.when(wid == _NW - 1)
    def _():
        w0 = pltpu.async_copy(block_v, out_hbm.at[0, :, pl.ds(_B - _BPW, _BPW + 1)], sem0)
        w1 = pltpu.async_copy(block_v, out_hbm.at[1, :, pl.ds(_B - _BPW, _BPW + 1)], sem1)
        w0.wait()
        w1.wait()


def kernel(inputs, embed_weight):
    idx = inputs.reshape(-1).astype(jnp.int32)
    padded_idx = jnp.concatenate(
        [jnp.zeros((1,), jnp.int32), idx, jnp.zeros((15,), jnp.int32)]
    )
    out = _embed_lookup(padded_idx, embed_weight)
    return out.transpose(0, 2, 1).reshape(2, _N, 1, _D)


# speedup vs baseline: 2.6465x; 1.0248x over previous
import functools

import jax
import jax.numpy as jnp
from jax import lax
from jax.experimental import pallas as pl
from jax.experimental.pallas import tpu as pltpu
from jax.experimental.pallas import tpu_sc as plsc

_NC = 2
_NS = 16
_NW = _NC * _NS

_B = 16384
_N = _B + 1
_D = 64
_V = 32
_BPW = _B // _NW
_L = 16
_G = _BPW // _L

_mesh = plsc.VectorSubcoreMesh(
    core_axis_name="c", subcore_axis_name="s", num_cores=_NC, num_subcores=_NS
)


@functools.partial(
    pl.kernel,
    mesh=_mesh,
    out_type=jax.ShapeDtypeStruct((2, _D, _N), jnp.float32),
    compiler_params=pltpu.CompilerParams(needs_layout_passes=False),
    scratch_types=[
        pltpu.VMEM((_BPW + _L,), jnp.int32),
        pltpu.VMEM((_V, _D), jnp.float32),
        pltpu.VMEM((_D, _BPW + 1), jnp.float32),
        pltpu.SemaphoreType.DMA,
        pltpu.SemaphoreType.DMA,
    ],
)
def _embed_lookup(idx_hbm, table_hbm, out_hbm, idx_v, table_v, block_v, sem0, sem1):
    wid = lax.axis_index("s") * _NC + lax.axis_index("c")
    base = wid * _BPW
    cp_idx = pltpu.async_copy(idx_hbm.at[pl.ds(base, _BPW + _L)], idx_v, sem0)
    cp_tbl = pltpu.async_copy(table_hbm, table_v, sem1)
    cp_idx.wait()
    cp_tbl.wait()

    @plsc.parallel_loop(0, _G, 1)
    def _(g):
        col = g * _L
        idx_vec = idx_v[pl.ds(col, _L)]
        for d0 in range(0, _D, 8):
            vals = [
                plsc.load_gather(table_v, [idx_vec, jnp.full((_L,), d, jnp.int32)])
                for d in range(d0, d0 + 8)
            ]
            for k in range(8):
                block_v[d0 + k, pl.ds(col, _L)] = vals[k]

    @pl.when(wid == _NW - 1)
    def _():
        idx_vec = idx_v[pl.ds(_BPW, _L)]
        lane0 = lax.iota(jnp.int32, _L) == 0
        for d in range(_D):
            colv = jnp.full((_L,), d, jnp.int32)
            vals = plsc.load_gather(table_v, [idx_vec, colv])
            plsc.store_scatter(
                block_v,
                [colv, jnp.full((_L,), _BPW, jnp.int32)],
                vals,
                mask=lane0,
            )

    @pl.when(wid < _NW - 1)
    def _():
        w0 = pltpu.async_copy(
            block_v.at[:, pl.ds(0, _BPW)], out_hbm.at[0, :, pl.ds(base, _BPW)], sem0
        )
        w1 = pltpu.async_copy(
            block_v.at[:, pl.ds(0, _BPW)], out_hbm.at[1, :, pl.ds(base, _BPW)], sem1
        )
        w0.wait()
        w1.wait()

    @pl.when(wid == _NW - 1)
    def _():
        w0 = pltpu.async_copy(block_v, out_hbm.at[0, :, pl.ds(_B - _BPW, _BPW + 1)], sem0)
        w1 = pltpu.async_copy(block_v, out_hbm.at[1, :, pl.ds(_B - _BPW, _BPW + 1)], sem1)
        w0.wait()
        w1.wait()


def kernel(inputs, embed_weight):
    idx = inputs.reshape(-1).astype(jnp.int32)
    padded_idx = jnp.concatenate(
        [jnp.zeros((1,), jnp.int32), idx, jnp.zeros((15,), jnp.int32)]
    )
    out = _embed_lookup(padded_idx, embed_weight)
    return out.transpose(0, 2, 1).reshape(2, _N, 1, _D)


# speedup vs baseline: 3.7866x; 1.4308x over previous
import functools

import jax
import jax.numpy as jnp
from jax import lax
from jax.experimental import pallas as pl
from jax.experimental.pallas import tpu as pltpu
from jax.experimental.pallas import tpu_sc as plsc

_NC = 2
_NS = 16
_NW = _NC * _NS

_B = 16384
_N = _B + 1
_D = 64
_V = 32
_BPW = _B // _NW
_L = 16
_G = _BPW // _L
_TS = _D + 1

_mesh = plsc.VectorSubcoreMesh(
    core_axis_name="c", subcore_axis_name="s", num_cores=_NC, num_subcores=_NS
)


@functools.partial(
    pl.kernel,
    mesh=_mesh,
    out_type=jax.ShapeDtypeStruct((2, _D, _N), jnp.float32),
    compiler_params=pltpu.CompilerParams(needs_layout_passes=False),
    scratch_types=[
        pltpu.VMEM((_BPW + _L,), jnp.int32),
        pltpu.VMEM((_V, _D), jnp.float32),
        pltpu.VMEM((_V * _TS,), jnp.float32),
        pltpu.VMEM((_D, _BPW + 1), jnp.float32),
        pltpu.SemaphoreType.DMA,
        pltpu.SemaphoreType.DMA,
    ],
)
def _embed_lookup(
    idx_hbm, table_hbm, out_hbm, idx_v, tstage_v, table_v, block_v, sem0, sem1
):
    wid = lax.axis_index("s") * _NC + lax.axis_index("c")
    base = wid * _BPW
    cp_idx = pltpu.async_copy(idx_hbm.at[pl.ds(base, _BPW + _L)], idx_v, sem0)
    cp_tbl = pltpu.async_copy(table_hbm, tstage_v, sem1)
    cp_idx.wait()
    cp_tbl.wait()

    for i in range(_V):
        rvals = [tstage_v[i, pl.ds(16 * k, _L)] for k in range(_D // _L)]
        for k in range(_D // _L):
            table_v[pl.ds(i * _TS + 16 * k, _L)] = rvals[k]

    def gather_groups(g_lo, g_hi):
        @plsc.parallel_loop(g_lo, g_hi, 1)
        def _(g):
            col = g * _L
            idx65 = idx_v[pl.ds(col, _L)] * _TS
            for d0 in range(0, _D, 8):
                vals = [
                    plsc.load_gather(table_v, [idx65 + d])
                    for d in range(d0, d0 + 8)
                ]
                for k in range(8):
                    block_v[d0 + k, pl.ds(col, _L)] = vals[k]

    _H = _BPW // 2
    is_tail = wid == _NW - 1

    gather_groups(0, _G // 2)

    @pl.when(jnp.logical_not(is_tail))
    def _():
        wa0 = pltpu.async_copy(
            block_v.at[:, pl.ds(0, _H)], out_hbm.at[0, :, pl.ds(base, _H)], sem0
        )
        wa1 = pltpu.async_copy(
            block_v.at[:, pl.ds(0, _H)], out_hbm.at[1, :, pl.ds(base, _H)], sem1
        )

    gather_groups(_G // 2, _G)

    @pl.when(wid == _NW - 1)
    def _():
        idx65 = idx_v[pl.ds(_BPW, _L)] * _TS
        lane0 = lax.iota(jnp.int32, _L) == 0
        for d in range(_D):
            vals = plsc.load_gather(table_v, [idx65 + d])
            plsc.store_scatter(
                block_v,
                [jnp.full((_L,), d, jnp.int32), jnp.full((_L,), _BPW, jnp.int32)],
                vals,
                mask=lane0,
            )

    @pl.when(jnp.logical_not(is_tail))
    def _():
        wb0 = pltpu.async_copy(
            block_v.at[:, pl.ds(_H, _H)], out_hbm.at[0, :, pl.ds(base + _H, _H)], sem0
        )
        wb1 = pltpu.async_copy(
            block_v.at[:, pl.ds(_H, _H)], out_hbm.at[1, :, pl.ds(base + _H, _H)], sem1
        )
        wb0.wait()
        wb1.wait()
        wb0.wait()
        wb1.wait()

    @pl.when(is_tail)
    def _():
        wt0 = pltpu.async_copy(block_v, out_hbm.at[0, :, pl.ds(_B - _BPW, _BPW + 1)], sem0)
        wt1 = pltpu.async_copy(block_v, out_hbm.at[1, :, pl.ds(_B - _BPW, _BPW + 1)], sem1)
        wt0.wait()
        wt1.wait()


def kernel(inputs, embed_weight):
    idx = inputs.reshape(-1).astype(jnp.int32)
    padded_idx = jnp.concatenate(
        [jnp.zeros((1,), jnp.int32), idx, jnp.zeros((15,), jnp.int32)]
    )
    out = _embed_lookup(padded_idx, embed_weight)
    return out.transpose(0, 2, 1).reshape(2, _N, 1, _D)
